# baseline (speedup 1.0000x reference)
import functools

import jax
import jax.numpy as jnp
from jax import lax
from jax.experimental import pallas as pl
from jax.experimental.pallas import tpu as pltpu
from jax.experimental.pallas import tpu_sc as plsc

LANES = 16


def _sc_pooled_sum(x_chunks, emb_table, *, n_chunks_per_row, chunk):
    total_chunks, _ = x_chunks.shape
    B = total_chunks // n_chunks_per_row
    V, D = emb_table.shape
    n_groups = D // LANES

    mesh = plsc.VectorSubcoreMesh(core_axis_name="c", subcore_axis_name="s")
    NC, NS = mesh.num_cores, mesh.num_subcores
    NW = NC * NS
    rows_per_w = B // NW
    chunks_per_w = rows_per_w * n_chunks_per_row

    UNROLL = 4
    assert chunk % UNROLL == 0

    @functools.partial(
        pl.kernel,
        out_type=jax.ShapeDtypeStruct((B, D), jnp.float32),
        mesh=mesh,
        compiler_params=pltpu.CompilerParams(use_tc_tiling_on_sc=False),
        scratch_types=[
            pltpu.VMEM((chunks_per_w, chunk), jnp.int32),
            pltpu.VMEM((chunk, D), jnp.float32),
            pltpu.VMEM((rows_per_w, D), jnp.float32),
        ],
    )
    def sc_kernel(idx_hbm, table_hbm, out_hbm, idx_v, buf_v, out_v):
        c = lax.axis_index("c")
        s = lax.axis_index("s")
        wid = s * NC + c
        row0 = wid * rows_per_w
        chunk0 = wid * chunks_per_w

        pltpu.sync_copy(idx_hbm.at[pl.ds(chunk0, chunks_per_w)], idx_v)

        @pl.loop(0, rows_per_w)
        def _(i):
            accs = tuple(jnp.zeros((LANES,), jnp.float32) for _ in range(n_groups))
            for cpr in range(n_chunks_per_row):
                j = i * n_chunks_per_row + cpr
                pltpu.sync_copy(table_hbm.at[idx_v.at[j]], buf_v)

                def step(r, a, _cpr=cpr):
                    a = list(a)
                    for rr in range(UNROLL):
                        row = r * UNROLL + rr
                        for g in range(n_groups):
                            a[g] = a[g] + buf_v[row, pl.ds(g * LANES, LANES)]
                    return tuple(a)

                accs = lax.fori_loop(0, chunk // UNROLL, step, accs)
            for g in range(n_groups):
                out_v[i, pl.ds(g * LANES, LANES)] = accs[g]

        pltpu.sync_copy(out_v, out_hbm.at[pl.ds(row0, rows_per_w)])

    return sc_kernel(x_chunks, emb_table)


def _tc_head(pooled, W, b2, label2, *, seq_len):
    B, D = pooled.shape
    C = W.shape[0]

    def head_kernel(p_ref, w_ref, b_ref, l_ref, o_ref):
        y = jnp.dot(p_ref[...], w_ref[...].T,
                    preferred_element_type=jnp.float32)
        y = y * (1.0 / seq_len) + b_ref[...]
        m = jnp.max(y, axis=1, keepdims=True)
        lse = jnp.log(jnp.sum(jnp.exp(y - m), axis=1, keepdims=True)) + m
        onehot = lax.broadcasted_iota(jnp.int32, y.shape, 1) == l_ref[...]
        ylab = jnp.sum(jnp.where(onehot, y, 0.0), axis=1, keepdims=True)
        o_ref[...] = jnp.sum(lse - ylab, axis=0, keepdims=True) * (1.0 / B)

    return pl.pallas_call(
        head_kernel,
        out_shape=jax.ShapeDtypeStruct((1, 1), jnp.float32),
    )(pooled, W, b2, label2)


def kernel(x, label, emb_table, W, b):
    B, S = x.shape
    V, D = emb_table.shape

    n_chunks_per_row = -(-S // 128)
    assert S % n_chunks_per_row == 0
    chunk = S // n_chunks_per_row
    x_chunks = x.reshape(B * n_chunks_per_row, chunk)

    pooled = _sc_pooled_sum(x_chunks, emb_table,
                            n_chunks_per_row=n_chunks_per_row, chunk=chunk)

    loss = _tc_head(pooled, W, b.reshape(1, -1).astype(jnp.float32),
                    label.reshape(B, 1).astype(jnp.int32), seq_len=S)
    return loss.reshape(())


# speedup vs baseline: 1.1723x; 1.1723x over previous
import functools

import jax
import jax.numpy as jnp
from jax import lax
from jax.experimental import pallas as pl
from jax.experimental.pallas import tpu as pltpu
from jax.experimental.pallas import tpu_sc as plsc

LANES = 16


def _sc_pooled_sum(x_chunks, emb_table, *, n_chunks_per_row, chunk):
    total_chunks, _ = x_chunks.shape
    B = total_chunks // n_chunks_per_row
    V, D = emb_table.shape
    n_groups = D // LANES

    mesh = plsc.VectorSubcoreMesh(core_axis_name="c", subcore_axis_name="s")
    NC, NS = mesh.num_cores, mesh.num_subcores
    NW = NC * NS
    rows_per_w = B // NW
    chunks_per_w = rows_per_w * n_chunks_per_row

    UNROLL = 4
    assert chunk % UNROLL == 0

    @functools.partial(
        pl.kernel,
        out_type=jax.ShapeDtypeStruct((B, D), jnp.float32),
        mesh=mesh,
        compiler_params=pltpu.CompilerParams(use_tc_tiling_on_sc=False),
        scratch_types=[
            pltpu.VMEM((chunks_per_w, chunk), jnp.int32),
            pltpu.VMEM((chunk, D), jnp.float32),
            pltpu.VMEM((chunk, D), jnp.float32),
            pltpu.VMEM((rows_per_w, D), jnp.float32),
            pltpu.SemaphoreType.DMA,
            pltpu.SemaphoreType.DMA,
        ],
    )
    def sc_kernel(idx_hbm, table_hbm, out_hbm, idx_v, buf0_v, buf1_v,
                  out_v, sem0, sem1):
        c = lax.axis_index("c")
        s = lax.axis_index("s")
        wid = s * NC + c
        row0 = wid * rows_per_w
        chunk0 = wid * chunks_per_w

        pltpu.sync_copy(idx_hbm.at[pl.ds(chunk0, chunks_per_w)], idx_v)

        def start_gather(j, buf, sem):
            pltpu.make_async_copy(table_hbm.at[idx_v.at[j]], buf, sem).start()

        def wait_gather(j, buf, sem):
            pltpu.make_async_copy(table_hbm.at[idx_v.at[j]], buf, sem).wait()

        def accumulate(buf, accs):
            def step(r, a):
                a = list(a)
                for rr in range(UNROLL):
                    row = r * UNROLL + rr
                    for g in range(n_groups):
                        a[g] = a[g] + buf[row, pl.ds(g * LANES, LANES)]
                return tuple(a)

            return lax.fori_loop(0, chunk // UNROLL, step, accs)

        start_gather(0, buf0_v, sem0)

        @pl.loop(0, rows_per_w)
        def _(i):
            j0 = i * 2
            start_gather(j0 + 1, buf1_v, sem1)
            wait_gather(j0, buf0_v, sem0)
            accs = tuple(jnp.zeros((LANES,), jnp.float32)
                         for _ in range(n_groups))
            accs = accumulate(buf0_v, accs)

            @pl.when(i < rows_per_w - 1)
            def _():
                start_gather(j0 + 2, buf0_v, sem0)

            wait_gather(j0 + 1, buf1_v, sem1)
            accs = accumulate(buf1_v, accs)
            for g in range(n_groups):
                out_v[i, pl.ds(g * LANES, LANES)] = accs[g]

        pltpu.sync_copy(out_v, out_hbm.at[pl.ds(row0, rows_per_w)])

    return sc_kernel(x_chunks, emb_table)


def _tc_head(pooled, W, b2, label2, *, seq_len):
    B, D = pooled.shape
    C = W.shape[0]

    def head_kernel(p_ref, w_ref, b_ref, l_ref, o_ref):
        y = jnp.dot(p_ref[...], w_ref[...].T,
                    preferred_element_type=jnp.float32)
        y = y * (1.0 / seq_len) + b_ref[...]
        m = jnp.max(y, axis=1, keepdims=True)
        lse = jnp.log(jnp.sum(jnp.exp(y - m), axis=1, keepdims=True)) + m
        onehot = lax.broadcasted_iota(jnp.int32, y.shape, 1) == l_ref[...]
        ylab = jnp.sum(jnp.where(onehot, y, 0.0), axis=1, keepdims=True)
        o_ref[...] = jnp.sum(lse - ylab, axis=0, keepdims=True) * (1.0 / B)

    return pl.pallas_call(
        head_kernel,
        out_shape=jax.ShapeDtypeStruct((1, 1), jnp.float32),
    )(pooled, W, b2, label2)


def kernel(x, label, emb_table, W, b):
    B, S = x.shape
    V, D = emb_table.shape

    n_chunks_per_row = -(-S // 128)
    assert S % n_chunks_per_row == 0
    chunk = S // n_chunks_per_row
    x_chunks = x.reshape(B * n_chunks_per_row, chunk)

    pooled = _sc_pooled_sum(x_chunks, emb_table,
                            n_chunks_per_row=n_chunks_per_row, chunk=chunk)

    loss = _tc_head(pooled, W, b.reshape(1, -1).astype(jnp.float32),
                    label.reshape(B, 1).astype(jnp.int32), seq_len=S)
    return loss.reshape(())


# speedup vs baseline: 2.0290x; 1.7309x over previous
import functools

import jax
import jax.numpy as jnp
from jax import lax
from jax.experimental import pallas as pl
from jax.experimental.pallas import tpu as pltpu
from jax.experimental.pallas import tpu_sc as plsc

LANES = 16


def _sc_pooled_sum(x_chunks, emb_table, *, n_chunks_per_row, chunk):
    total_chunks, _ = x_chunks.shape
    B = total_chunks // n_chunks_per_row
    V, D = emb_table.shape
    n_groups = D // LANES

    mesh = plsc.VectorSubcoreMesh(core_axis_name="c", subcore_axis_name="s")
    NC, NS = mesh.num_cores, mesh.num_subcores
    NW = NC * NS
    rows_per_w = B // NW
    chunks_per_w = rows_per_w * n_chunks_per_row

    UNROLL = 4
    assert chunk % UNROLL == 0

    @functools.partial(
        pl.kernel,
        out_type=jax.ShapeDtypeStruct((B, D), jnp.float32),
        mesh=mesh,
        compiler_params=pltpu.CompilerParams(use_tc_tiling_on_sc=False),
        scratch_types=[
            pltpu.VMEM((chunks_per_w, chunk), jnp.int32),
            pltpu.VMEM((chunk, D), jnp.float32),
            pltpu.VMEM((chunk, D), jnp.float32),
            pltpu.VMEM((rows_per_w, D), jnp.float32),
            pltpu.SemaphoreType.DMA,
            pltpu.SemaphoreType.DMA,
        ],
    )
    def sc_kernel(idx_hbm, table_hbm, out_hbm, idx_v, buf0_v, buf1_v,
                  out_v, sem0, sem1):
        c = lax.axis_index("c")
        s = lax.axis_index("s")
        wid = s * NC + c
        row0 = wid * rows_per_w
        chunk0 = wid * chunks_per_w

        pltpu.sync_copy(idx_hbm.at[pl.ds(chunk0, chunks_per_w)], idx_v)

        def start_gather(j, buf, sem):
            pltpu.make_async_copy(table_hbm.at[idx_v.at[j]], buf, sem).start()

        def wait_gather(j, buf, sem):
            pltpu.make_async_copy(table_hbm.at[idx_v.at[j]], buf, sem).wait()

        def accumulate(buf, accs):
            def step(r, a):
                a = list(a)
                for rr in range(UNROLL):
                    row = r * UNROLL + rr
                    for g in range(n_groups):
                        a[g] = a[g] + buf[row, pl.ds(g * LANES, LANES)]
                return tuple(a)

            return lax.fori_loop(0, chunk // UNROLL, step, accs)

        start_gather(0, buf0_v, sem0)

        @pl.loop(0, rows_per_w)
        def _(i):
            j0 = i * 2
            start_gather(j0 + 1, buf1_v, sem1)
            wait_gather(j0, buf0_v, sem0)
            accs = tuple(jnp.zeros((LANES,), jnp.float32)
                         for _ in range(n_groups))
            accs = accumulate(buf0_v, accs)

            @pl.when(i < rows_per_w - 1)
            def _():
                start_gather(j0 + 2, buf0_v, sem0)

            wait_gather(j0 + 1, buf1_v, sem1)
            accs = accumulate(buf1_v, accs)
            for g in range(n_groups):
                out_v[i, pl.ds(g * LANES, LANES)] = accs[g]

        pltpu.sync_copy(out_v, out_hbm.at[pl.ds(row0, rows_per_w)])

    return sc_kernel(x_chunks, emb_table)


FOLD_PAIR = 4096


def _tc_fold(tT):
    D, V = tT.shape
    H = V // 2
    P = FOLD_PAIR
    grid = -(-V // (2 * P))

    def fold_kernel(in_ref, o_ref):
        o_ref[:, 0:D] = in_ref[:, 0:P].T
        o_ref[:, D:2 * D] = in_ref[:, P:2 * P].T

    return pl.pallas_call(
        fold_kernel,
        grid=(grid,),
        in_specs=[pl.BlockSpec((D, 2 * P), lambda i: (0, i))],
        out_specs=pl.BlockSpec((P, 2 * D), lambda i: (i, 0)),
        out_shape=jax.ShapeDtypeStruct((H, 2 * D), jnp.float32),
    )(tT)


def _tc_head(pooled, W, b2, label2, *, seq_len):
    B, D = pooled.shape
    C = W.shape[0]

    def head_kernel(p_ref, w_ref, b_ref, l_ref, o_ref):
        y = jnp.dot(p_ref[...], w_ref[...].T,
                    preferred_element_type=jnp.float32)
        y = y * (1.0 / seq_len) + b_ref[...]
        m = jnp.max(y, axis=1, keepdims=True)
        lse = jnp.log(jnp.sum(jnp.exp(y - m), axis=1, keepdims=True)) + m
        onehot = lax.broadcasted_iota(jnp.int32, y.shape, 1) == l_ref[...]
        ylab = jnp.sum(jnp.where(onehot, y, 0.0), axis=1, keepdims=True)
        o_ref[...] = jnp.sum(lse - ylab, axis=0, keepdims=True) * (1.0 / B)

    return pl.pallas_call(
        head_kernel,
        out_shape=jax.ShapeDtypeStruct((1, 1), jnp.float32),
    )(pooled, W, b2, label2)


def kernel(x, label, emb_table, W, b):
    B, S = x.shape
    V, D = emb_table.shape

    folded = _tc_fold(emb_table.T)
    table_rm = folded.reshape(V, D)
    P = FOLD_PAIR
    x_remap = (x >> 13 << 13) + 2 * (x & (P - 1)) + ((x >> 12) & 1)

    n_chunks_per_row = -(-S // 128)
    assert S % n_chunks_per_row == 0
    chunk = S // n_chunks_per_row
    x_chunks = x_remap.reshape(B * n_chunks_per_row, chunk)

    pooled = _sc_pooled_sum(x_chunks, table_rm,
                            n_chunks_per_row=n_chunks_per_row, chunk=chunk)

    loss = _tc_head(pooled, W, b.reshape(1, -1).astype(jnp.float32),
                    label.reshape(B, 1).astype(jnp.int32), seq_len=S)
    return loss.reshape(())


# speedup vs baseline: 2.5480x; 1.2558x over previous
import functools

import jax
import jax.numpy as jnp
from jax import lax
from jax.experimental import pallas as pl
from jax.experimental.pallas import tpu as pltpu
from jax.experimental.pallas import tpu_sc as plsc

LANES = 16


def _sc_pooled_sum(x_chunks, emb_table, *, n_chunks_per_row, chunk):
    total_chunks, _ = x_chunks.shape
    B = total_chunks // n_chunks_per_row
    V, D = emb_table.shape
    n_groups = D // LANES

    mesh = plsc.VectorSubcoreMesh(core_axis_name="c", subcore_axis_name="s")
    NC, NS = mesh.num_cores, mesh.num_subcores
    NW = NC * NS
    rows_per_w = B // NW
    chunks_per_w = rows_per_w * n_chunks_per_row

    UNROLL = 4
    assert chunk % UNROLL == 0

    NBUF = 4
    assert n_chunks_per_row == 2 and rows_per_w % 2 == 0

    @functools.partial(
        pl.kernel,
        out_type=jax.ShapeDtypeStruct((B, D), jnp.float32),
        mesh=mesh,
        compiler_params=pltpu.CompilerParams(use_tc_tiling_on_sc=False),
        scratch_types=(
            [pltpu.VMEM((chunks_per_w, chunk), jnp.int32)]
            + [pltpu.VMEM((chunk, D), jnp.float32) for _ in range(NBUF)]
            + [pltpu.VMEM((rows_per_w, D), jnp.float32)]
            + [pltpu.SemaphoreType.DMA for _ in range(NBUF)]
        ),
    )
    def sc_kernel(idx_hbm, table_hbm, out_hbm, idx_v, b0, b1, b2, b3,
                  out_v, s0, s1, s2, s3):
        bufs = [b0, b1, b2, b3]
        sems = [s0, s1, s2, s3]
        c = lax.axis_index("c")
        s = lax.axis_index("s")
        wid = s * NC + c
        row0 = wid * rows_per_w
        chunk0 = wid * chunks_per_w

        pltpu.sync_copy(idx_hbm.at[pl.ds(chunk0, chunks_per_w)], idx_v)

        def start_gather(j, buf, sem):
            pltpu.make_async_copy(table_hbm.at[idx_v.at[j]], buf, sem).start()

        def wait_gather(j, buf, sem):
            pltpu.make_async_copy(table_hbm.at[idx_v.at[j]], buf, sem).wait()

        def accumulate(buf, accs):
            def step(r, a):
                a = list(a)
                for rr in range(UNROLL):
                    row = r * UNROLL + rr
                    for g in range(n_groups):
                        a[g] = a[g] + buf[row, pl.ds(g * LANES, LANES)]
                return tuple(a)

            return lax.fori_loop(0, chunk // UNROLL, step, accs)

        for cc in range(NBUF - 1):
            start_gather(cc, bufs[cc], sems[cc])

        zeros = tuple(jnp.zeros((LANES,), jnp.float32)
                      for _ in range(n_groups))

        @pl.loop(0, rows_per_w // 2)
        def _(i2):
            j0 = i2 * NBUF
            accs = zeros
            for cc in range(NBUF):
                j = j0 + cc
                wait_gather(j, bufs[cc], sems[cc])
                nb = (cc + NBUF - 1) % NBUF

                @pl.when(j + NBUF - 1 < chunks_per_w)
                def _(j=j, nb=nb):
                    start_gather(j + NBUF - 1, bufs[nb], sems[nb])

                accs = accumulate(bufs[cc], accs)
                if cc % n_chunks_per_row == n_chunks_per_row - 1:
                    i = i2 * 2 + cc // n_chunks_per_row
                    for g in range(n_groups):
                        out_v[i, pl.ds(g * LANES, LANES)] = accs[g]
                    accs = zeros

        pltpu.sync_copy(out_v, out_hbm.at[pl.ds(row0, rows_per_w)])

    return sc_kernel(x_chunks, emb_table)


FOLD_PAIR = 8192


def _tc_fold(tT):
    D, V = tT.shape
    H = V // 2
    P = FOLD_PAIR
    grid = -(-V // (2 * P))

    def fold_kernel(in_ref, o_ref):
        eye = (lax.broadcasted_iota(jnp.int32, (D, D), 0)
               == lax.broadcasted_iota(jnp.int32, (D, D), 1)
               ).astype(jnp.float32)
        ot = lax.dot_general(in_ref[...], eye, (((0,), (0,)), ((), ())),
                             preferred_element_type=jnp.float32)
        o_ref[:, 0:D] = ot[0:P]
        o_ref[:, D:2 * D] = ot[P:2 * P]

    return pl.pallas_call(
        fold_kernel,
        grid=(grid,),
        in_specs=[pl.BlockSpec((D, 2 * P), lambda i: (0, i))],
        out_specs=pl.BlockSpec((P, 2 * D), lambda i: (i, 0)),
        out_shape=jax.ShapeDtypeStruct((H, 2 * D), jnp.float32),
        compiler_params=pltpu.CompilerParams(
            fuse_transposed_lhs_in_matmul=True),
    )(tT)


def _tc_head(pooled, W, b2, label2, *, seq_len):
    B, D = pooled.shape
    C = W.shape[0]

    def head_kernel(p_ref, w_ref, b_ref, l_ref, o_ref):
        y = jnp.dot(p_ref[...], w_ref[...].T,
                    preferred_element_type=jnp.float32)
        y = y * (1.0 / seq_len) + b_ref[...]
        m = jnp.max(y, axis=1, keepdims=True)
        lse = jnp.log(jnp.sum(jnp.exp(y - m), axis=1, keepdims=True)) + m
        onehot = lax.broadcasted_iota(jnp.int32, y.shape, 1) == l_ref[...]
        ylab = jnp.sum(jnp.where(onehot, y, 0.0), axis=1, keepdims=True)
        o_ref[...] = jnp.sum(lse - ylab, axis=0, keepdims=True) * (1.0 / B)

    return pl.pallas_call(
        head_kernel,
        out_shape=jax.ShapeDtypeStruct((1, 1), jnp.float32),
    )(pooled, W, b2, label2)


def kernel(x, label, emb_table, W, b):
    B, S = x.shape
    V, D = emb_table.shape

    folded = _tc_fold(emb_table.T)
    table_rm = folded.reshape(V, D)
    P = FOLD_PAIR
    x_remap = (x & ~(2 * P - 1)) + 2 * (x & (P - 1)) + ((x // P) & 1)

    n_chunks_per_row = -(-S // 128)
    assert S % n_chunks_per_row == 0
    chunk = S // n_chunks_per_row
    x_chunks = x_remap.reshape(B * n_chunks_per_row, chunk)

    pooled = _sc_pooled_sum(x_chunks, table_rm,
                            n_chunks_per_row=n_chunks_per_row, chunk=chunk)

    loss = _tc_head(pooled, W, b.reshape(1, -1).astype(jnp.float32),
                    label.reshape(B, 1).astype(jnp.int32), seq_len=S)
    return loss.reshape(())


# speedup vs baseline: 3.0441x; 1.1947x over previous
import functools

import jax
import jax.numpy as jnp
from jax import lax
from jax.experimental import pallas as pl
from jax.experimental.pallas import tpu as pltpu
from jax.experimental.pallas import tpu_sc as plsc

LANES = 16


def _sc_pooled_sum(x_chunks, emb_table, *, n_chunks_per_row, chunk):
    total_chunks, _ = x_chunks.shape
    B = total_chunks // n_chunks_per_row
    V, D = emb_table.shape
    n_groups = D // LANES

    mesh = plsc.VectorSubcoreMesh(core_axis_name="c", subcore_axis_name="s")
    NC, NS = mesh.num_cores, mesh.num_subcores
    NW = NC * NS
    rows_per_w = B // NW
    chunks_per_w = rows_per_w * n_chunks_per_row

    UNROLL = 4
    assert chunk % UNROLL == 0

    NBUF = 4
    assert n_chunks_per_row == 2 and rows_per_w % 2 == 0

    @functools.partial(
        pl.kernel,
        out_type=jax.ShapeDtypeStruct((B, D), jnp.float32),
        mesh=mesh,
        compiler_params=pltpu.CompilerParams(use_tc_tiling_on_sc=False),
        scratch_types=(
            [pltpu.VMEM((chunks_per_w, chunk), jnp.int32)]
            + [pltpu.VMEM((chunk, D), jnp.float32) for _ in range(NBUF)]
            + [pltpu.VMEM((rows_per_w, D), jnp.float32)]
            + [pltpu.SemaphoreType.DMA for _ in range(NBUF)]
        ),
    )
    def sc_kernel(idx_hbm, table_hbm, out_hbm, idx_v, b0, b1, b2, b3,
                  out_v, s0, s1, s2, s3):
        bufs = [b0, b1, b2, b3]
        sems = [s0, s1, s2, s3]
        c = lax.axis_index("c")
        s = lax.axis_index("s")
        wid = s * NC + c
        row0 = wid * rows_per_w
        chunk0 = wid * chunks_per_w

        pltpu.sync_copy(idx_hbm.at[pl.ds(chunk0, chunks_per_w)], idx_v)

        def start_gather(j, buf, sem):
            pltpu.make_async_copy(table_hbm.at[idx_v.at[j]], buf, sem).start()

        def wait_gather(j, buf, sem):
            pltpu.make_async_copy(table_hbm.at[idx_v.at[j]], buf, sem).wait()

        def accumulate(buf, accs):
            def step(r, a):
                a = list(a)
                for rr in range(UNROLL):
                    row = r * UNROLL + rr
                    for g in range(n_groups):
                        a[g] = a[g] + buf[row, pl.ds(g * LANES, LANES)]
                return tuple(a)

            return lax.fori_loop(0, chunk // UNROLL, step, accs)

        for cc in range(NBUF - 1):
            start_gather(cc, bufs[cc], sems[cc])

        zeros = tuple(jnp.zeros((LANES,), jnp.float32)
                      for _ in range(n_groups))

        @pl.loop(0, rows_per_w // 2)
        def _(i2):
            j0 = i2 * NBUF
            accs = zeros
            for cc in range(NBUF):
                j = j0 + cc
                wait_gather(j, bufs[cc], sems[cc])
                nb = (cc + NBUF - 1) % NBUF

                @pl.when(j + NBUF - 1 < chunks_per_w)
                def _(j=j, nb=nb):
                    start_gather(j + NBUF - 1, bufs[nb], sems[nb])

                accs = accumulate(bufs[cc], accs)
                if cc % n_chunks_per_row == n_chunks_per_row - 1:
                    i = i2 * 2 + cc // n_chunks_per_row
                    for g in range(n_groups):
                        out_v[i, pl.ds(g * LANES, LANES)] = accs[g]
                    accs = zeros

        pltpu.sync_copy(out_v, out_hbm.at[pl.ds(row0, rows_per_w)])

    return sc_kernel(x_chunks, emb_table)


FOLD_PAIR = 8192


def _tc_fold(tT):
    D, V = tT.shape
    H = V // 2
    P = FOLD_PAIR
    grid = -(-V // (2 * P))

    def fold_kernel(in_ref, o_ref):
        s = jnp.concatenate([in_ref[:, 0:P], in_ref[:, P:2 * P]], axis=0)
        o_ref[...] = s.T

    return pl.pallas_call(
        fold_kernel,
        grid=(grid,),
        in_specs=[pl.BlockSpec((D, 2 * P), lambda i: (0, i))],
        out_specs=pl.BlockSpec((P, 2 * D), lambda i: (i, 0)),
        out_shape=jax.ShapeDtypeStruct((H, 2 * D), jnp.float32),
        compiler_params=pltpu.CompilerParams(
            fuse_transposed_lhs_in_matmul=True),
    )(tT)


def _tc_head(pooled, W, b2, label2, *, seq_len):
    B, D = pooled.shape
    C = W.shape[0]

    def head_kernel(p_ref, w_ref, b_ref, l_ref, o_ref):
        y = jnp.dot(p_ref[...], w_ref[...].T,
                    preferred_element_type=jnp.float32)
        y = y * (1.0 / seq_len) + b_ref[...]
        m = jnp.max(y, axis=1, keepdims=True)
        lse = jnp.log(jnp.sum(jnp.exp(y - m), axis=1, keepdims=True)) + m
        onehot = lax.broadcasted_iota(jnp.int32, y.shape, 1) == l_ref[...]
        ylab = jnp.sum(jnp.where(onehot, y, 0.0), axis=1, keepdims=True)
        o_ref[...] = jnp.sum(lse - ylab, axis=0, keepdims=True) * (1.0 / B)

    return pl.pallas_call(
        head_kernel,
        out_shape=jax.ShapeDtypeStruct((1, 1), jnp.float32),
    )(pooled, W, b2, label2)


def kernel(x, label, emb_table, W, b):
    B, S = x.shape
    V, D = emb_table.shape

    folded = _tc_fold(emb_table.T)
    table_rm = folded.reshape(V, D)
    P = FOLD_PAIR
    x_remap = (x & ~(2 * P - 1)) + 2 * (x & (P - 1)) + ((x // P) & 1)

    n_chunks_per_row = -(-S // 128)
    assert S % n_chunks_per_row == 0
    chunk = S // n_chunks_per_row
    x_chunks = x_remap.reshape(B * n_chunks_per_row, chunk)

    pooled = _sc_pooled_sum(x_chunks, table_rm,
                            n_chunks_per_row=n_chunks_per_row, chunk=chunk)

    loss = _tc_head(pooled, W, b.reshape(1, -1).astype(jnp.float32),
                    label.reshape(B, 1).astype(jnp.int32), seq_len=S)
    return loss.reshape(())
